# initial kernel scaffold (unmeasured)
import jax
import jax.numpy as jnp
from jax import lax
from jax.experimental import pallas as pl
from jax.experimental.pallas import tpu as pltpu

N_DEV = 32
B = 256
D = 256
CH = B // N_DEV


def kernel(x, Win0, Wout0, Win1, Wout1, Win2, Wout2):
    def body(
        x_ref, win0_ref, wout0_ref, win1_ref, wout1_ref, win2_ref, wout2_ref,
        out_ref,
        part_buf,
        red_buf,
        rs_comm,
        ag_comm,
        send_sems,
        rs_recv_sems,
        ag_recv_sems,
    ):
        my = lax.axis_index("i")

        barrier = pltpu.get_barrier_semaphore()
        for k in range(1, N_DEV):
            pl.semaphore_signal(
                barrier, inc=1,
                device_id=((my + k) % N_DEV,),
                device_id_type=pl.DeviceIdType.MESH,
            )
        pl.semaphore_wait(barrier, N_DEV - 1)

        def reduce_scatter(part):
            part3 = part.reshape(N_DEV, CH, D)
            part_buf[...] = part3
            rs_comm[my] = lax.dynamic_index_in_dim(part3, my, 0, keepdims=False)
            sends = []
            for k in range(1, N_DEV):
                dest = (my + k) % N_DEV
                d = pltpu.make_async_remote_copy(
                    src_ref=part_buf.at[dest],
                    dst_ref=rs_comm.at[my],
                    send_sem=send_sems.at[k - 1],
                    recv_sem=rs_recv_sems.at[my],
                    device_id=(dest,),
                    device_id_type=pl.DeviceIdType.MESH,
                )
                d.start()
                sends.append(d)
            for k in range(1, N_DEV):
                src = (my + (N_DEV - k)) % N_DEV
                r = pltpu.make_async_remote_copy(
                    src_ref=rs_comm.at[src],
                    dst_ref=rs_comm.at[src],
                    send_sem=send_sems.at[0],
                    recv_sem=rs_recv_sems.at[src],
                    device_id=(my,),
                    device_id_type=pl.DeviceIdType.MESH,
                )
                r.wait_recv()
            for d in sends:
                d.wait_send()
            return jnp.sum(rs_comm[...], axis=0)

        def all_gather(red):
            red_buf[...] = red
            ag_comm[my] = red
            sends = []
            for k in range(1, N_DEV):
                dest = (my + k) % N_DEV
                d = pltpu.make_async_remote_copy(
                    src_ref=red_buf,
                    dst_ref=ag_comm.at[my],
                    send_sem=send_sems.at[k - 1],
                    recv_sem=ag_recv_sems.at[my],
                    device_id=(dest,),
                    device_id_type=pl.DeviceIdType.MESH,
                )
                d.start()
                sends.append(d)
            for k in range(1, N_DEV):
                src = (my + (N_DEV - k)) % N_DEV
                r = pltpu.make_async_remote_copy(
                    src_ref=ag_comm.at[src],
                    dst_ref=ag_comm.at[src],
                    send_sem=send_sems.at[0],
                    recv_sem=ag_recv_sems.at[src],
                    device_id=(my,),
                    device_id_type=pl.DeviceIdType.MESH,
                )
                r.wait_recv()
            for d in sends:
                d.wait_send()
            return ag_comm[...].reshape(B, D)

        def layer(x_val, win_ref, wout_ref):
            xb = x_val.astype(jnp.bfloat16)
            h = jnp.dot(xb, win_ref[...].astype(jnp.bfloat16),
                        preferred_element_type=jnp.float32)
            h = jnp.maximum(h, 0.0)
            return jnp.dot(h.astype(jnp.bfloat16),
                           wout_ref[...].astype(jnp.bfloat16),
                           preferred_element_type=jnp.float32)

        x_val = x_ref[...]
        part = layer(x_val, win0_ref, wout0_ref)
        x_val = all_gather(reduce_scatter(part))
        part = layer(x_val, win1_ref, wout1_ref)
        x_val = all_gather(reduce_scatter(part))
        part = layer(x_val, win2_ref, wout2_ref)
        out_ref[...] = reduce_scatter(part)

    return pl.pallas_call(
        body,
        out_shape=jax.ShapeDtypeStruct((CH, D), jnp.float32),
        in_specs=[pl.BlockSpec(memory_space=pltpu.VMEM)] * 7,
        out_specs=pl.BlockSpec(memory_space=pltpu.VMEM),
        scratch_shapes=[
            pltpu.VMEM((N_DEV, CH, D), jnp.float32),
            pltpu.VMEM((CH, D), jnp.float32),
            pltpu.VMEM((N_DEV, CH, D), jnp.float32),
            pltpu.VMEM((N_DEV, CH, D), jnp.float32),
            pltpu.SemaphoreType.DMA((N_DEV - 1,)),
            pltpu.SemaphoreType.DMA((N_DEV,)),
            pltpu.SemaphoreType.DMA((N_DEV,)),
        ],
        compiler_params=pltpu.CompilerParams(collective_id=0),
    )(x, Win0, Wout0, Win1, Wout1, Win2, Wout2)


# baseline (device time: 46610 ns/iter reference)
import jax
import jax.numpy as jnp
from jax import lax
from jax.experimental import pallas as pl
from jax.experimental.pallas import tpu as pltpu

N_DEV = 32
B = 256
D = 256
CH = B // N_DEV


def kernel(x, Win0, Wout0, Win1, Wout1, Win2, Wout2):
    def body(
        x_ref, win0_ref, wout0_ref, win1_ref, wout1_ref, win2_ref, wout2_ref,
        out_ref,
        part_buf,
        red_buf,
        rs_comm,
        ag_comm,
        send_sems,
        rs_recv_sems,
        ag_recv_sems,
    ):
        my = lax.axis_index("i")

        barrier = pltpu.get_barrier_semaphore()
        for k in range(1, N_DEV):
            pl.semaphore_signal(
                barrier, inc=1,
                device_id=((my + k) % N_DEV,),
                device_id_type=pl.DeviceIdType.MESH,
            )
        pl.semaphore_wait(barrier, N_DEV - 1)

        def reduce_scatter(part):
            part3 = part.reshape(N_DEV, CH, D)
            part_buf[...] = part3
            rs_comm[my] = part_buf[my]
            sends = []
            for k in range(1, N_DEV):
                dest = (my + k) % N_DEV
                d = pltpu.make_async_remote_copy(
                    src_ref=part_buf.at[dest],
                    dst_ref=rs_comm.at[my],
                    send_sem=send_sems.at[k - 1],
                    recv_sem=rs_recv_sems.at[my],
                    device_id=(dest,),
                    device_id_type=pl.DeviceIdType.MESH,
                )
                d.start()
                sends.append(d)
            for k in range(1, N_DEV):
                src = (my + (N_DEV - k)) % N_DEV
                r = pltpu.make_async_remote_copy(
                    src_ref=rs_comm.at[src],
                    dst_ref=rs_comm.at[src],
                    send_sem=send_sems.at[0],
                    recv_sem=rs_recv_sems.at[src],
                    device_id=(my,),
                    device_id_type=pl.DeviceIdType.MESH,
                )
                r.wait_recv()
            for d in sends:
                d.wait_send()
            return jnp.sum(rs_comm[...], axis=0)

        def all_gather(red):
            red_buf[...] = red
            ag_comm[my] = red
            sends = []
            for k in range(1, N_DEV):
                dest = (my + k) % N_DEV
                d = pltpu.make_async_remote_copy(
                    src_ref=red_buf,
                    dst_ref=ag_comm.at[my],
                    send_sem=send_sems.at[k - 1],
                    recv_sem=ag_recv_sems.at[my],
                    device_id=(dest,),
                    device_id_type=pl.DeviceIdType.MESH,
                )
                d.start()
                sends.append(d)
            for k in range(1, N_DEV):
                src = (my + (N_DEV - k)) % N_DEV
                r = pltpu.make_async_remote_copy(
                    src_ref=ag_comm.at[src],
                    dst_ref=ag_comm.at[src],
                    send_sem=send_sems.at[0],
                    recv_sem=ag_recv_sems.at[src],
                    device_id=(my,),
                    device_id_type=pl.DeviceIdType.MESH,
                )
                r.wait_recv()
            for d in sends:
                d.wait_send()
            return ag_comm[...].reshape(B, D)

        def layer(x_val, win_ref, wout_ref):
            xb = x_val.astype(jnp.bfloat16)
            h = jnp.dot(xb, win_ref[...].astype(jnp.bfloat16),
                        preferred_element_type=jnp.float32)
            h = jnp.maximum(h, 0.0)
            return jnp.dot(h.astype(jnp.bfloat16),
                           wout_ref[...].astype(jnp.bfloat16),
                           preferred_element_type=jnp.float32)

        x_val = x_ref[...]
        part = layer(x_val, win0_ref, wout0_ref)
        x_val = all_gather(reduce_scatter(part))
        part = layer(x_val, win1_ref, wout1_ref)
        x_val = all_gather(reduce_scatter(part))
        part = layer(x_val, win2_ref, wout2_ref)
        out_ref[...] = reduce_scatter(part)

    return pl.pallas_call(
        body,
        out_shape=jax.ShapeDtypeStruct((CH, D), jnp.float32),
        in_specs=[pl.BlockSpec(memory_space=pltpu.VMEM)] * 7,
        out_specs=pl.BlockSpec(memory_space=pltpu.VMEM),
        scratch_shapes=[
            pltpu.VMEM((N_DEV, CH, D), jnp.float32),
            pltpu.VMEM((CH, D), jnp.float32),
            pltpu.VMEM((N_DEV, CH, D), jnp.float32),
            pltpu.VMEM((N_DEV, CH, D), jnp.float32),
            pltpu.SemaphoreType.DMA((N_DEV - 1,)),
            pltpu.SemaphoreType.DMA((N_DEV,)),
            pltpu.SemaphoreType.DMA((N_DEV,)),
        ],
        compiler_params=pltpu.CompilerParams(collective_id=0),
    )(x, Win0, Wout0, Win1, Wout1, Win2, Wout2)


# device time: 42017 ns/iter; 1.1093x vs baseline; 1.1093x over previous
import jax
import jax.numpy as jnp
from jax import lax
from jax.experimental import pallas as pl
from jax.experimental.pallas import tpu as pltpu

N_DEV = 32
B = 256
D = 256
CH = B // N_DEV


def kernel(x, Win0, Wout0, Win1, Wout1, Win2, Wout2):
    def body(
        x_ref, win0_ref, wout0_ref, win1_ref, wout1_ref, win2_ref, wout2_ref,
        out_ref,
        part_buf,
        red_buf,
        rs_comm,
        ag_comm,
        send_sems,
        rs_recv_sems,
        ag_recv_sems,
    ):
        my = lax.axis_index("i")

        def reduce_scatter(part):
            part3 = part.reshape(N_DEV, CH, D)
            part_buf[...] = part3.astype(jnp.bfloat16)
            rs_comm[my] = part_buf[my]
            sends = []
            for k in range(1, N_DEV):
                dest = (my + k) % N_DEV
                d = pltpu.make_async_remote_copy(
                    src_ref=part_buf.at[dest],
                    dst_ref=rs_comm.at[my],
                    send_sem=send_sems.at[k - 1],
                    recv_sem=rs_recv_sems.at[my],
                    device_id=(dest,),
                    device_id_type=pl.DeviceIdType.MESH,
                )
                d.start()
                sends.append(d)
            for k in range(1, N_DEV):
                src = (my + (N_DEV - k)) % N_DEV
                r = pltpu.make_async_remote_copy(
                    src_ref=rs_comm.at[src],
                    dst_ref=rs_comm.at[src],
                    send_sem=send_sems.at[0],
                    recv_sem=rs_recv_sems.at[src],
                    device_id=(my,),
                    device_id_type=pl.DeviceIdType.MESH,
                )
                r.wait_recv()
            for d in sends:
                d.wait_send()
            return jnp.sum(rs_comm[...].astype(jnp.float32), axis=0)

        def all_gather(red):
            red_buf[...] = red.astype(jnp.bfloat16)
            ag_comm[my] = red_buf[...]
            sends = []
            for k in range(1, N_DEV):
                dest = (my + k) % N_DEV
                d = pltpu.make_async_remote_copy(
                    src_ref=red_buf,
                    dst_ref=ag_comm.at[my],
                    send_sem=send_sems.at[k - 1],
                    recv_sem=ag_recv_sems.at[my],
                    device_id=(dest,),
                    device_id_type=pl.DeviceIdType.MESH,
                )
                d.start()
                sends.append(d)
            for k in range(1, N_DEV):
                src = (my + (N_DEV - k)) % N_DEV
                r = pltpu.make_async_remote_copy(
                    src_ref=ag_comm.at[src],
                    dst_ref=ag_comm.at[src],
                    send_sem=send_sems.at[0],
                    recv_sem=ag_recv_sems.at[src],
                    device_id=(my,),
                    device_id_type=pl.DeviceIdType.MESH,
                )
                r.wait_recv()
            for d in sends:
                d.wait_send()
            return ag_comm[...].reshape(B, D)

        def layer(x_val, win_ref, wout_ref):
            xb = x_val.astype(jnp.bfloat16)
            h = jnp.dot(xb, win_ref[...].astype(jnp.bfloat16),
                        preferred_element_type=jnp.float32)
            h = jnp.maximum(h, 0.0)
            return jnp.dot(h.astype(jnp.bfloat16),
                           wout_ref[...].astype(jnp.bfloat16),
                           preferred_element_type=jnp.float32)

        barrier = pltpu.get_barrier_semaphore()
        for k in range(1, N_DEV):
            pl.semaphore_signal(
                barrier, inc=1,
                device_id=((my + k) % N_DEV,),
                device_id_type=pl.DeviceIdType.MESH,
            )
        part = layer(x_ref[...], win0_ref, wout0_ref)
        pl.semaphore_wait(barrier, N_DEV - 1)
        x_val = all_gather(reduce_scatter(part))
        part = layer(x_val, win1_ref, wout1_ref)
        x_val = all_gather(reduce_scatter(part))
        part = layer(x_val, win2_ref, wout2_ref)
        out_ref[...] = reduce_scatter(part)

    return pl.pallas_call(
        body,
        out_shape=jax.ShapeDtypeStruct((CH, D), jnp.float32),
        in_specs=[pl.BlockSpec(memory_space=pltpu.VMEM)] * 7,
        out_specs=pl.BlockSpec(memory_space=pltpu.VMEM),
        scratch_shapes=[
            pltpu.VMEM((N_DEV, CH, D), jnp.bfloat16),
            pltpu.VMEM((CH, D), jnp.bfloat16),
            pltpu.VMEM((N_DEV, CH, D), jnp.bfloat16),
            pltpu.VMEM((N_DEV, CH, D), jnp.bfloat16),
            pltpu.SemaphoreType.DMA((N_DEV - 1,)),
            pltpu.SemaphoreType.DMA((N_DEV,)),
            pltpu.SemaphoreType.DMA((N_DEV,)),
        ],
        compiler_params=pltpu.CompilerParams(collective_id=0),
    )(x, Win0, Wout0, Win1, Wout1, Win2, Wout2)


# device time: 41301 ns/iter; 1.1285x vs baseline; 1.0173x over previous
import jax
import jax.numpy as jnp
from jax import lax
from jax.experimental import pallas as pl
from jax.experimental.pallas import tpu as pltpu

N_DEV = 32
HALF = N_DEV // 2
B = 256
D = 256
CH = B // N_DEV


def kernel(x, Win0, Wout0, Win1, Wout1, Win2, Wout2):
    def body(
        x_ref, win0_ref, wout0_ref, win1_ref, wout1_ref, win2_ref, wout2_ref,
        out_ref,
        part_buf,
        red_buf,
        rs_comm,
        ag_comm,
        rs_send_sems,
        ag_send_sems,
        rs_recv_sems,
        ag_recv_sems,
    ):
        my = lax.axis_index("i")

        def rs_send_desc(s):
            return pltpu.make_async_remote_copy(
                src_ref=part_buf.at[s],
                dst_ref=rs_comm.at[my],
                send_sem=rs_send_sems.at[s],
                recv_sem=rs_recv_sems.at[my],
                device_id=(s,),
                device_id_type=pl.DeviceIdType.MESH,
            )

        def rs_recv_desc(s):
            return pltpu.make_async_remote_copy(
                src_ref=rs_comm.at[s],
                dst_ref=rs_comm.at[s],
                send_sem=rs_send_sems.at[s],
                recv_sem=rs_recv_sems.at[s],
                device_id=(my,),
                device_id_type=pl.DeviceIdType.MESH,
            )

        def ag_send_desc(s):
            return pltpu.make_async_remote_copy(
                src_ref=red_buf,
                dst_ref=ag_comm.at[my],
                send_sem=ag_send_sems.at[s],
                recv_sem=ag_recv_sems.at[my],
                device_id=(s,),
                device_id_type=pl.DeviceIdType.MESH,
            )

        def ag_recv_desc(s):
            return pltpu.make_async_remote_copy(
                src_ref=ag_comm.at[s],
                dst_ref=ag_comm.at[s],
                send_sem=ag_send_sems.at[s],
                recv_sem=ag_recv_sems.at[s],
                device_id=(my,),
                device_id_type=pl.DeviceIdType.MESH,
            )

        def for_peers(lo, hi, fn):
            for s in range(lo, hi):
                @pl.when(s != my)
                def _(s=s):
                    fn(s)

        def rs_sends(lo, hi):
            for_peers(lo, hi, lambda s: rs_send_desc(s).start())

        def rs_finish(lo, hi):
            for_peers(lo, hi, lambda s: rs_recv_desc(s).wait_recv())
            for_peers(lo, hi, lambda s: rs_send_desc(s).wait_send())

        def ag_sends(lo, hi):
            for_peers(lo, hi, lambda s: ag_send_desc(s).start())

        def ag_waits(lo, hi):
            for_peers(lo, hi, lambda s: ag_recv_desc(s).wait_recv())

        def ag_wait_sends():
            for_peers(0, N_DEV, lambda s: ag_send_desc(s).wait_send())

        def mlp_rows(xv, win_ref, wout_ref):
            h = jnp.dot(xv.astype(jnp.bfloat16),
                        win_ref[...].astype(jnp.bfloat16),
                        preferred_element_type=jnp.float32)
            h = jnp.maximum(h, 0.0)
            return jnp.dot(h.astype(jnp.bfloat16),
                           wout_ref[...].astype(jnp.bfloat16),
                           preferred_element_type=jnp.float32)

        def store_part(lo, hi, val):
            part_buf[lo:hi] = val.reshape(hi - lo, CH, D).astype(jnp.bfloat16)

        barrier = pltpu.get_barrier_semaphore()
        for k in range(1, N_DEV):
            pl.semaphore_signal(
                barrier, inc=1,
                device_id=((my + k) % N_DEV,),
                device_id_type=pl.DeviceIdType.MESH,
            )

        x0 = x_ref[...]
        store_part(0, HALF, mlp_rows(x0[0:HALF * CH], win0_ref, wout0_ref))
        pl.semaphore_wait(barrier, N_DEV - 1)
        rs_sends(0, HALF)
        store_part(HALF, N_DEV, mlp_rows(x0[HALF * CH:], win0_ref, wout0_ref))
        rs_sends(HALF, N_DEV)
        rs_comm[my] = part_buf[my]

        for win_ref, wout_ref in ((win1_ref, wout1_ref), (win2_ref, wout2_ref)):
            rs_finish(0, N_DEV)
            red = jnp.sum(rs_comm[...].astype(jnp.float32), axis=0)
            red_buf[...] = red.astype(jnp.bfloat16)
            ag_comm[my] = red_buf[...]
            ag_sends(0, N_DEV)
            ag_waits(0, HALF)
            store_part(0, HALF, mlp_rows(
                ag_comm[0:HALF].reshape(HALF * CH, D), win_ref, wout_ref))
            rs_sends(0, HALF)
            ag_waits(HALF, N_DEV)
            store_part(HALF, N_DEV, mlp_rows(
                ag_comm[HALF:N_DEV].reshape(HALF * CH, D), win_ref, wout_ref))
            rs_sends(HALF, N_DEV)
            rs_comm[my] = part_buf[my]
            ag_wait_sends()

        rs_finish(0, N_DEV)
        out_ref[...] = jnp.sum(rs_comm[...].astype(jnp.float32), axis=0)

    return pl.pallas_call(
        body,
        out_shape=jax.ShapeDtypeStruct((CH, D), jnp.float32),
        in_specs=[pl.BlockSpec(memory_space=pltpu.VMEM)] * 7,
        out_specs=pl.BlockSpec(memory_space=pltpu.VMEM),
        scratch_shapes=[
            pltpu.VMEM((N_DEV, CH, D), jnp.bfloat16),
            pltpu.VMEM((CH, D), jnp.bfloat16),
            pltpu.VMEM((N_DEV, CH, D), jnp.bfloat16),
            pltpu.VMEM((N_DEV, CH, D), jnp.bfloat16),
            pltpu.SemaphoreType.DMA((N_DEV,)),
            pltpu.SemaphoreType.DMA((N_DEV,)),
            pltpu.SemaphoreType.DMA((N_DEV,)),
            pltpu.SemaphoreType.DMA((N_DEV,)),
        ],
        compiler_params=pltpu.CompilerParams(collective_id=0),
    )(x, Win0, Wout0, Win1, Wout1, Win2, Wout2)
